# baseline (speedup 1.0000x reference)
import functools

import jax
import jax.numpy as jnp
from jax import lax
from jax.experimental import pallas as pl
from jax.experimental.pallas import tpu as pltpu

B, N, C, E = 4, 1024, 512, 4
K = 512
G = 8
SL = 8


def _score_topk_pool_kernel(x_ref, adj_ref, w1_ref, b1_ref, w2_ref, b2_ref,
                            w3_ref, b3_ref, pf_ref, pa_ref, idx_ref):
    x = x_ref[0]
    h = jnp.maximum(jnp.dot(x, w1_ref[...], preferred_element_type=jnp.float32)
                    + b1_ref[...], 0.0)
    h = jnp.maximum(jnp.dot(h, w2_ref[...], preferred_element_type=jnp.float32)
                    + b2_ref[...], 0.0)
    s = jnp.dot(h, w3_ref[...], preferred_element_type=jnp.float32) + b3_ref[...]
    s_row = s.reshape(1, N)

    ii = lax.broadcasted_iota(jnp.int32, (N, N), 0)
    jj = lax.broadcasted_iota(jnp.int32, (N, N), 1)
    beats = jnp.where((s > s_row) | ((s == s_row) & (ii < jj)), 1.0, 0.0)
    rank_row = jnp.sum(beats, axis=0, keepdims=True)
    rank_col = (N - 1.0) - jnp.sum(beats, axis=1, keepdims=True)

    r_col = lax.broadcasted_iota(jnp.int32, (K, N), 0).astype(jnp.float32)
    P = jnp.where(rank_row == r_col, 1.0, 0.0)
    r_row = lax.broadcasted_iota(jnp.int32, (N, K), 1).astype(jnp.float32)
    PT = jnp.where(rank_col == r_row, 1.0, 0.0)

    i_col = lax.broadcasted_iota(jnp.int32, (N, K), 0).astype(jnp.float32)
    idx_f = jnp.sum(PT * i_col, axis=0, keepdims=True)
    idx_ref[0] = idx_f.astype(jnp.int32)

    pf_ref[0] = jnp.dot(P, x, precision=lax.Precision.DEFAULT,
                        preferred_element_type=jnp.float32)
    ap = jnp.dot(adj_ref[0], PT, precision=lax.Precision.DEFAULT,
                 preferred_element_type=jnp.float32)
    pa_ref[0] = jnp.dot(P, ap, precision=lax.Precision.DEFAULT,
                        preferred_element_type=jnp.float32)


NSEM = 8
W = 256


def _edge_copy_kernel(idx_ref, ef_ref, out_ref, sems):
    def cp(i):
        b = i // K
        r = i - b * K
        return pltpu.make_async_copy(
            ef_ref.at[b, idx_ref[b, r]], out_ref.at[b, r], sems.at[i % NSEM])

    def head(i, _):
        cp(i).start()
        return 0

    lax.fori_loop(0, W, head, 0)

    def steady(i, _):
        cp(i).start()
        cp(i - W).wait()
        return 0

    lax.fori_loop(W, B * K, steady, 0)

    def tail(i, _):
        cp(i).wait()
        return 0

    lax.fori_loop(B * K - W, B * K, tail, 0)


@jax.jit
def kernel(x, adjacency, edge_features, superpoint_centroids,
           W1, b1, W2, b2, W3, b3):
    pf, pa, idx3 = pl.pallas_call(
        _score_topk_pool_kernel,
        grid=(B,),
        in_specs=[
            pl.BlockSpec((1, N, C), lambda b: (b, 0, 0)),
            pl.BlockSpec((1, N, N), lambda b: (b, 0, 0)),
            pl.BlockSpec((C, 64), lambda b: (0, 0)),
            pl.BlockSpec((1, 64), lambda b: (0, 0)),
            pl.BlockSpec((64, 16), lambda b: (0, 0)),
            pl.BlockSpec((1, 16), lambda b: (0, 0)),
            pl.BlockSpec((16, 1), lambda b: (0, 0)),
            pl.BlockSpec((1, 1), lambda b: (0, 0)),
        ],
        out_specs=[
            pl.BlockSpec((1, K, C), lambda b: (b, 0, 0)),
            pl.BlockSpec((1, K, K), lambda b: (b, 0, 0)),
            pl.BlockSpec((1, 1, K), lambda b: (b, 0, 0)),
        ],
        out_shape=[
            jax.ShapeDtypeStruct((B, K, C), jnp.float32),
            jax.ShapeDtypeStruct((B, K, K), jnp.float32),
            jax.ShapeDtypeStruct((B, 1, K), jnp.int32),
        ],
    )(x, adjacency, W1, b1.reshape(1, 64), W2, b2.reshape(1, 16),
      W3, b3.reshape(1, 1))

    idx = idx3.reshape(B, K)

    eft = jnp.transpose(edge_features, (0, 1, 3, 2))
    peft = pl.pallas_call(
        _edge_copy_kernel,
        in_specs=[
            pl.BlockSpec(memory_space=pltpu.SMEM),
            pl.BlockSpec(memory_space=pl.ANY),
        ],
        out_specs=pl.BlockSpec(memory_space=pl.ANY),
        out_shape=jax.ShapeDtypeStruct((B, K, E, N), jnp.float32),
        scratch_shapes=[pltpu.SemaphoreType.DMA((NSEM,))],
    )(idx, eft)
    pooled_edge_features = jnp.transpose(peft, (0, 1, 3, 2))

    return (pf, pa, pooled_edge_features)


# speedup vs baseline: 12.2128x; 12.2128x over previous
import functools

import jax
import jax.numpy as jnp
from jax import lax
from jax.experimental import pallas as pl
from jax.experimental.pallas import tpu as pltpu

B, N, C, E = 4, 1024, 512, 4
K = 512
G = 32
SL = 8


def _score_topk_pool_kernel(x_ref, adj_ref, w1_ref, b1_ref, w2_ref, b2_ref,
                            w3_ref, b3_ref, pf_ref, pa_ref, idx_ref):
    x = x_ref[0]
    h = jnp.maximum(jnp.dot(x, w1_ref[...], preferred_element_type=jnp.float32)
                    + b1_ref[...], 0.0)
    h = jnp.maximum(jnp.dot(h, w2_ref[...], preferred_element_type=jnp.float32)
                    + b2_ref[...], 0.0)
    s = jnp.dot(h, w3_ref[...], preferred_element_type=jnp.float32) + b3_ref[...]
    s_row = s.reshape(1, N)

    ii = lax.broadcasted_iota(jnp.int32, (N, N), 0)
    jj = lax.broadcasted_iota(jnp.int32, (N, N), 1)
    beats = jnp.where((s > s_row) | ((s == s_row) & (ii < jj)), 1.0, 0.0)
    rank_row = jnp.sum(beats, axis=0, keepdims=True)
    rank_col = (N - 1.0) - jnp.sum(beats, axis=1, keepdims=True)

    r_col = lax.broadcasted_iota(jnp.int32, (K, N), 0).astype(jnp.float32)
    P = jnp.where(rank_row == r_col, 1.0, 0.0)
    r_row = lax.broadcasted_iota(jnp.int32, (N, K), 1).astype(jnp.float32)
    PT = jnp.where(rank_col == r_row, 1.0, 0.0)

    i_col = lax.broadcasted_iota(jnp.int32, (N, K), 0).astype(jnp.float32)
    idx_f = jnp.sum(PT * i_col, axis=0, keepdims=True)
    idx_ref[0] = idx_f.astype(jnp.int32)

    pf_ref[0] = jnp.dot(P, x, precision=lax.Precision.DEFAULT,
                        preferred_element_type=jnp.float32)
    ap = jnp.dot(adj_ref[0], PT, precision=lax.Precision.DEFAULT,
                 preferred_element_type=jnp.float32)
    pa_ref[0] = jnp.dot(P, ap, precision=lax.Precision.DEFAULT,
                        preferred_element_type=jnp.float32)


def _edge_gather_kernel(idx_ref, *refs):
    in_refs = refs[:G]
    out_ref = refs[G]
    for g in range(G):
        out_ref[0, g] = in_refs[g][0, 0]


def _edge_in_map(g, b, r, idx_ref):
    return (b, idx_ref[b, r * G + g], 0, 0)


def _edge_out_map(b, r, idx_ref):
    return (b, r, 0, 0)


@jax.jit
def kernel(x, adjacency, edge_features, superpoint_centroids,
           W1, b1, W2, b2, W3, b3):
    pf, pa, idx3 = pl.pallas_call(
        _score_topk_pool_kernel,
        grid=(B,),
        in_specs=[
            pl.BlockSpec((1, N, C), lambda b: (b, 0, 0)),
            pl.BlockSpec((1, N, N), lambda b: (b, 0, 0)),
            pl.BlockSpec((C, 64), lambda b: (0, 0)),
            pl.BlockSpec((1, 64), lambda b: (0, 0)),
            pl.BlockSpec((64, 16), lambda b: (0, 0)),
            pl.BlockSpec((1, 16), lambda b: (0, 0)),
            pl.BlockSpec((16, 1), lambda b: (0, 0)),
            pl.BlockSpec((1, 1), lambda b: (0, 0)),
        ],
        out_specs=[
            pl.BlockSpec((1, K, C), lambda b: (b, 0, 0)),
            pl.BlockSpec((1, K, K), lambda b: (b, 0, 0)),
            pl.BlockSpec((1, 1, K), lambda b: (b, 0, 0)),
        ],
        out_shape=[
            jax.ShapeDtypeStruct((B, K, C), jnp.float32),
            jax.ShapeDtypeStruct((B, K, K), jnp.float32),
            jax.ShapeDtypeStruct((B, 1, K), jnp.int32),
        ],
    )(x, adjacency, W1, b1.reshape(1, 64), W2, b2.reshape(1, 16),
      W3, b3.reshape(1, 1))

    idx = idx3.reshape(B, K)

    eft = jnp.transpose(edge_features, (0, 1, 3, 2))
    peft = pl.pallas_call(
        _edge_gather_kernel,
        grid_spec=pltpu.PrefetchScalarGridSpec(
            num_scalar_prefetch=1,
            grid=(B, K // G),
            in_specs=[
                pl.BlockSpec((1, 1, E, N), functools.partial(_edge_in_map, g))
                for g in range(G)
            ],
            out_specs=pl.BlockSpec((1, G, E, N), _edge_out_map),
        ),
        out_shape=jax.ShapeDtypeStruct((B, K, E, N), jnp.float32),
    )(idx, *([eft] * G))
    pooled_edge_features = jnp.transpose(peft, (0, 1, 3, 2))

    return (pf, pa, pooled_edge_features)


# speedup vs baseline: 12.6422x; 1.0352x over previous
import functools

import jax
import jax.numpy as jnp
from jax import lax
from jax.experimental import pallas as pl
from jax.experimental.pallas import tpu as pltpu

B, N, C, E = 4, 1024, 512, 4
K = 512
G = 64
SL = 8


def _score_topk_pool_kernel(x_ref, adj_ref, w1_ref, b1_ref, w2_ref, b2_ref,
                            w3_ref, b3_ref, pf_ref, pa_ref, idx_ref):
    x = x_ref[0]
    h = jnp.maximum(jnp.dot(x, w1_ref[...], preferred_element_type=jnp.float32)
                    + b1_ref[...], 0.0)
    h = jnp.maximum(jnp.dot(h, w2_ref[...], preferred_element_type=jnp.float32)
                    + b2_ref[...], 0.0)
    s = jnp.dot(h, w3_ref[...], preferred_element_type=jnp.float32) + b3_ref[...]
    s_row = s.reshape(1, N)

    ii = lax.broadcasted_iota(jnp.int32, (N, N), 0)
    jj = lax.broadcasted_iota(jnp.int32, (N, N), 1)
    beats = jnp.where((s > s_row) | ((s == s_row) & (ii < jj)), 1.0, 0.0)
    rank_row = jnp.sum(beats, axis=0, keepdims=True)
    rank_col = (N - 1.0) - jnp.sum(beats, axis=1, keepdims=True)

    r_col = lax.broadcasted_iota(jnp.int32, (K, N), 0).astype(jnp.float32)
    P = jnp.where(rank_row == r_col, 1.0, 0.0)
    r_row = lax.broadcasted_iota(jnp.int32, (N, K), 1).astype(jnp.float32)
    PT = jnp.where(rank_col == r_row, 1.0, 0.0)

    i_col = lax.broadcasted_iota(jnp.int32, (N, K), 0).astype(jnp.float32)
    idx_f = jnp.sum(PT * i_col, axis=0, keepdims=True)
    idx_ref[0] = idx_f.astype(jnp.int32)

    pf_ref[0] = jnp.dot(P, x, precision=lax.Precision.DEFAULT,
                        preferred_element_type=jnp.float32)
    ap = jnp.dot(adj_ref[0], PT, precision=lax.Precision.DEFAULT,
                 preferred_element_type=jnp.float32)
    pa_ref[0] = jnp.dot(P, ap, precision=lax.Precision.DEFAULT,
                        preferred_element_type=jnp.float32)


def _edge_gather_kernel(idx_ref, *refs):
    in_refs = refs[:G]
    out_ref = refs[G]
    for g in range(G):
        out_ref[0, g] = in_refs[g][0, 0]


def _edge_in_map(g, b, r, idx_ref):
    return (b, idx_ref[b, r * G + g], 0, 0)


def _edge_out_map(b, r, idx_ref):
    return (b, r, 0, 0)


@jax.jit
def kernel(x, adjacency, edge_features, superpoint_centroids,
           W1, b1, W2, b2, W3, b3):
    pf, pa, idx3 = pl.pallas_call(
        _score_topk_pool_kernel,
        grid=(B,),
        in_specs=[
            pl.BlockSpec((1, N, C), lambda b: (b, 0, 0)),
            pl.BlockSpec((1, N, N), lambda b: (b, 0, 0)),
            pl.BlockSpec((C, 64), lambda b: (0, 0)),
            pl.BlockSpec((1, 64), lambda b: (0, 0)),
            pl.BlockSpec((64, 16), lambda b: (0, 0)),
            pl.BlockSpec((1, 16), lambda b: (0, 0)),
            pl.BlockSpec((16, 1), lambda b: (0, 0)),
            pl.BlockSpec((1, 1), lambda b: (0, 0)),
        ],
        out_specs=[
            pl.BlockSpec((1, K, C), lambda b: (b, 0, 0)),
            pl.BlockSpec((1, K, K), lambda b: (b, 0, 0)),
            pl.BlockSpec((1, 1, K), lambda b: (b, 0, 0)),
        ],
        out_shape=[
            jax.ShapeDtypeStruct((B, K, C), jnp.float32),
            jax.ShapeDtypeStruct((B, K, K), jnp.float32),
            jax.ShapeDtypeStruct((B, 1, K), jnp.int32),
        ],
    )(x, adjacency, W1, b1.reshape(1, 64), W2, b2.reshape(1, 16),
      W3, b3.reshape(1, 1))

    idx = idx3.reshape(B, K)

    eft = jnp.transpose(edge_features, (0, 1, 3, 2))
    peft = pl.pallas_call(
        _edge_gather_kernel,
        grid_spec=pltpu.PrefetchScalarGridSpec(
            num_scalar_prefetch=1,
            grid=(B, K // G),
            in_specs=[
                pl.BlockSpec((1, 1, E, N), functools.partial(_edge_in_map, g))
                for g in range(G)
            ],
            out_specs=pl.BlockSpec((1, G, E, N), _edge_out_map),
        ),
        out_shape=jax.ShapeDtypeStruct((B, K, E, N), jnp.float32),
    )(idx, *([eft] * G))
    pooled_edge_features = jnp.transpose(peft, (0, 1, 3, 2))

    return (pf, pa, pooled_edge_features)


# speedup vs baseline: 14.1675x; 1.1206x over previous
import functools

import jax
import jax.numpy as jnp
from jax import lax
from jax.experimental import pallas as pl
from jax.experimental.pallas import tpu as pltpu
from jax.experimental.pallas import tpu_sc as plsc

B, N, C, E = 4, 1024, 512, 4
K = 512
G = 64
SL = 8


def _score_topk_pool_kernel(x_ref, adj_ref, w1_ref, b1_ref, w2_ref, b2_ref,
                            w3_ref, b3_ref, pf_ref, pa_ref, idx_ref):
    x = x_ref[0]
    h = jnp.maximum(jnp.dot(x, w1_ref[...], preferred_element_type=jnp.float32)
                    + b1_ref[...], 0.0)
    h = jnp.maximum(jnp.dot(h, w2_ref[...], preferred_element_type=jnp.float32)
                    + b2_ref[...], 0.0)
    s = jnp.dot(h, w3_ref[...], preferred_element_type=jnp.float32) + b3_ref[...]
    s_row = s.reshape(1, N)

    ii = lax.broadcasted_iota(jnp.int32, (N, N), 0)
    jj = lax.broadcasted_iota(jnp.int32, (N, N), 1)
    beats = jnp.where((s > s_row) | ((s == s_row) & (ii < jj)), 1.0, 0.0)
    rank_row = jnp.sum(beats, axis=0, keepdims=True)
    rank_col = (N - 1.0) - jnp.sum(beats, axis=1, keepdims=True)

    r_col = lax.broadcasted_iota(jnp.int32, (K, N), 0).astype(jnp.float32)
    P = jnp.where(rank_row == r_col, 1.0, 0.0)
    r_row = lax.broadcasted_iota(jnp.int32, (N, K), 1).astype(jnp.float32)
    PT = jnp.where(rank_col == r_row, 1.0, 0.0)

    i_col = lax.broadcasted_iota(jnp.int32, (N, K), 0).astype(jnp.float32)
    idx_f = jnp.sum(PT * i_col, axis=0, keepdims=True)
    idx_ref[0] = idx_f.astype(jnp.int32)

    pf_ref[0] = jnp.dot(P, x, precision=lax.Precision.DEFAULT,
                        preferred_element_type=jnp.float32)
    ap = jnp.dot(adj_ref[0], PT, precision=lax.Precision.DEFAULT,
                 preferred_element_type=jnp.float32)
    pa_ref[0] = jnp.dot(P, ap, precision=lax.Precision.DEFAULT,
                        preferred_element_type=jnp.float32)


CH = 8
RPW = (B * K) // 32


def _sc_edge_gather(idx_hbm, eft_hbm, out_hbm, idx_v, buf_v, sem):
    wid = lax.axis_index("s") * 2 + lax.axis_index("c")
    b = wid // (K // RPW)
    rbase = (wid % (K // RPW)) * RPW
    pltpu.sync_copy(idx_hbm.at[b, pl.ds(rbase, RPW)], idx_v)
    for c in range(RPW // CH):
        pltpu.async_copy(
            eft_hbm.at[b].at[idx_v.at[pl.ds(c * CH, CH)]], buf_v, sem,
        ).wait()
        pltpu.sync_copy(buf_v, out_hbm.at[b, pl.ds(rbase + c * CH, CH)])


def _edge_gather_sc(idx, eft):
    import functools as _ft
    mesh = plsc.VectorSubcoreMesh(core_axis_name="c", subcore_axis_name="s")
    return pl.kernel(
        _sc_edge_gather,
        mesh=mesh,
        out_type=jax.ShapeDtypeStruct((B, K, E, N), jnp.float32),
        scratch_types=[
            pltpu.VMEM((RPW,), jnp.int32),
            pltpu.VMEM((CH, E, N), jnp.float32),
            pltpu.SemaphoreType.DMA,
        ],
    )(idx, eft)


@jax.jit
def kernel(x, adjacency, edge_features, superpoint_centroids,
           W1, b1, W2, b2, W3, b3):
    pf, pa, idx3 = pl.pallas_call(
        _score_topk_pool_kernel,
        grid=(B,),
        in_specs=[
            pl.BlockSpec((1, N, C), lambda b: (b, 0, 0)),
            pl.BlockSpec((1, N, N), lambda b: (b, 0, 0)),
            pl.BlockSpec((C, 64), lambda b: (0, 0)),
            pl.BlockSpec((1, 64), lambda b: (0, 0)),
            pl.BlockSpec((64, 16), lambda b: (0, 0)),
            pl.BlockSpec((1, 16), lambda b: (0, 0)),
            pl.BlockSpec((16, 1), lambda b: (0, 0)),
            pl.BlockSpec((1, 1), lambda b: (0, 0)),
        ],
        out_specs=[
            pl.BlockSpec((1, K, C), lambda b: (b, 0, 0)),
            pl.BlockSpec((1, K, K), lambda b: (b, 0, 0)),
            pl.BlockSpec((1, 1, K), lambda b: (b, 0, 0)),
        ],
        out_shape=[
            jax.ShapeDtypeStruct((B, K, C), jnp.float32),
            jax.ShapeDtypeStruct((B, K, K), jnp.float32),
            jax.ShapeDtypeStruct((B, 1, K), jnp.int32),
        ],
    )(x, adjacency, W1, b1.reshape(1, 64), W2, b2.reshape(1, 16),
      W3, b3.reshape(1, 1))

    idx = idx3.reshape(B, K)

    eft = jnp.transpose(edge_features, (0, 1, 3, 2))
    peft = _edge_gather_sc(idx, eft)
    pooled_edge_features = jnp.transpose(peft, (0, 1, 3, 2))

    return (pf, pa, pooled_edge_features)


# speedup vs baseline: 14.9112x; 1.0525x over previous
import functools

import jax
import jax.numpy as jnp
from jax import lax
from jax.experimental import pallas as pl
from jax.experimental.pallas import tpu as pltpu
from jax.experimental.pallas import tpu_sc as plsc

B, N, C, E = 4, 1024, 512, 4
K = 512
G = 64
SL = 8


def _score_topk_pool_kernel(x_ref, adj_ref, w1_ref, b1_ref, w2_ref, b2_ref,
                            w3_ref, b3_ref, pf_ref, pa_ref, idx_ref):
    x = x_ref[0]
    h = jnp.maximum(jnp.dot(x, w1_ref[...], preferred_element_type=jnp.float32)
                    + b1_ref[...], 0.0)
    h = jnp.maximum(jnp.dot(h, w2_ref[...], preferred_element_type=jnp.float32)
                    + b2_ref[...], 0.0)
    s = jnp.dot(h, w3_ref[...], preferred_element_type=jnp.float32) + b3_ref[...]
    s_row = s.reshape(1, N)

    ii = lax.broadcasted_iota(jnp.int32, (N, N), 0)
    jj = lax.broadcasted_iota(jnp.int32, (N, N), 1)
    beats = jnp.where((s > s_row) | ((s == s_row) & (ii < jj)), 1.0, 0.0)
    rank_row = jnp.sum(beats, axis=0, keepdims=True)
    rank_col = (N - 1.0) - jnp.sum(beats, axis=1, keepdims=True)

    r_col = lax.broadcasted_iota(jnp.int32, (K, N), 0).astype(jnp.float32)
    P = jnp.where(rank_row == r_col, 1.0, 0.0)
    r_row = lax.broadcasted_iota(jnp.int32, (N, K), 1).astype(jnp.float32)
    PT = jnp.where(rank_col == r_row, 1.0, 0.0)

    i_col = lax.broadcasted_iota(jnp.int32, (N, K), 0).astype(jnp.float32)
    idx_f = jnp.sum(PT * i_col, axis=0, keepdims=True)
    idx_ref[0] = idx_f.astype(jnp.int32)

    pf_ref[0] = jnp.dot(P, x, precision=lax.Precision.DEFAULT,
                        preferred_element_type=jnp.float32)
    ap = jnp.dot(adj_ref[0], PT, precision=lax.Precision.DEFAULT,
                 preferred_element_type=jnp.float32)
    pa_ref[0] = jnp.dot(P, ap, precision=lax.Precision.DEFAULT,
                        preferred_element_type=jnp.float32)


CH = 16
RPW = (B * K) // 32


def _sc_edge_gather(idx_hbm, eft_hbm, out_hbm, idx_v, buf_v, sem):
    wid = lax.axis_index("s") * 2 + lax.axis_index("c")
    b = wid // (K // RPW)
    rbase = (wid % (K // RPW)) * RPW
    pltpu.sync_copy(idx_hbm.at[b, pl.ds(rbase, RPW)], idx_v)
    for c in range(RPW // CH):
        pltpu.async_copy(
            eft_hbm.at[b].at[idx_v.at[pl.ds(c * CH, CH)]], buf_v, sem,
        ).wait()
        pltpu.sync_copy(buf_v, out_hbm.at[b, pl.ds(rbase + c * CH, CH)])


def _edge_gather_sc(idx, eft):
    import functools as _ft
    mesh = plsc.VectorSubcoreMesh(core_axis_name="c", subcore_axis_name="s")
    return pl.kernel(
        _sc_edge_gather,
        mesh=mesh,
        out_type=jax.ShapeDtypeStruct((B, K, E, N), jnp.float32),
        scratch_types=[
            pltpu.VMEM((RPW,), jnp.int32),
            pltpu.VMEM((CH, E, N), jnp.float32),
            pltpu.SemaphoreType.DMA,
        ],
    )(idx, eft)


@jax.jit
def kernel(x, adjacency, edge_features, superpoint_centroids,
           W1, b1, W2, b2, W3, b3):
    pf, pa, idx3 = pl.pallas_call(
        _score_topk_pool_kernel,
        grid=(B,),
        in_specs=[
            pl.BlockSpec((1, N, C), lambda b: (b, 0, 0)),
            pl.BlockSpec((1, N, N), lambda b: (b, 0, 0)),
            pl.BlockSpec((C, 64), lambda b: (0, 0)),
            pl.BlockSpec((1, 64), lambda b: (0, 0)),
            pl.BlockSpec((64, 16), lambda b: (0, 0)),
            pl.BlockSpec((1, 16), lambda b: (0, 0)),
            pl.BlockSpec((16, 1), lambda b: (0, 0)),
            pl.BlockSpec((1, 1), lambda b: (0, 0)),
        ],
        out_specs=[
            pl.BlockSpec((1, K, C), lambda b: (b, 0, 0)),
            pl.BlockSpec((1, K, K), lambda b: (b, 0, 0)),
            pl.BlockSpec((1, 1, K), lambda b: (b, 0, 0)),
        ],
        out_shape=[
            jax.ShapeDtypeStruct((B, K, C), jnp.float32),
            jax.ShapeDtypeStruct((B, K, K), jnp.float32),
            jax.ShapeDtypeStruct((B, 1, K), jnp.int32),
        ],
    )(x, adjacency, W1, b1.reshape(1, 64), W2, b2.reshape(1, 16),
      W3, b3.reshape(1, 1))

    idx = idx3.reshape(B, K)

    eft = jnp.transpose(edge_features, (0, 1, 3, 2))
    peft = _edge_gather_sc(idx, eft)
    pooled_edge_features = jnp.transpose(peft, (0, 1, 3, 2))

    return (pf, pa, pooled_edge_features)
